# baseline (speedup 1.0000x reference)
import jax
import jax.numpy as jnp
from jax import lax
from jax.experimental import pallas as pl
from jax.experimental.pallas import tpu as pltpu
from jax.experimental.pallas import tpu_sc as plsc

N_NODES = 100000
NUM_TYPES = 128
L = 16
NW = 32
CHUNK = 512
NFULL = N_NODES // CHUNK
TAIL = N_NODES - NFULL * CHUNK
TAIL_BASE = NFULL * CHUNK
CHUNKS_PER_W = -(-NFULL // NW)

ROW_BLOCK = 20000


def _scatter_groups(buf, idx_v, n_rows, value):
    vals = jnp.full((L,), value, dtype=jnp.float32)
    row_off = lax.broadcasted_iota(jnp.int32, (L,), 0) * NUM_TYPES
    for g in range(n_rows // L):
        cols = idx_v[pl.ds(g * L, L)]
        flat = row_off + (g * L * NUM_TYPES) + cols
        plsc.store_scatter(buf, [flat], vals)


def _sc_body(elem_hbm, out_hbm, idx_v, buf, sem):
    wid = lax.axis_index("s") * 2 + lax.axis_index("c")

    zeros = jnp.zeros((L,), jnp.float32)

    def _zero_step(k, _):
        for j in range(8):
            buf[pl.ds(k * 8 * L + j * L, L)] = zeros
        return 0

    lax.fori_loop(0, CHUNK * NUM_TYPES // (8 * L), _zero_step, 0)

    def _do_chunk(base, n_rows):
        pltpu.sync_copy(elem_hbm.at[pl.ds(base, n_rows)], idx_v.at[pl.ds(0, n_rows)])
        _scatter_groups(buf, idx_v, n_rows, 1.0)
        pltpu.async_copy(buf.at[pl.ds(0, n_rows * NUM_TYPES)],
                         out_hbm.at[pl.ds(base * NUM_TYPES, n_rows * NUM_TYPES)],
                         sem).wait()
        _scatter_groups(buf, idx_v, n_rows, 0.0)

    def _chunk_step(i, _):
        c = wid + i * NW

        @pl.when(c < NFULL)
        def _():
            _do_chunk(c * CHUNK, CHUNK)

        return 0

    lax.fori_loop(0, CHUNKS_PER_W, _chunk_step, 0)

    @pl.when(wid == 3)
    def _():
        _do_chunk(TAIL_BASE, TAIL)


def _tc_body(idx_ref, out_ref):
    idx = idx_ref[...]
    classes = lax.broadcasted_iota(jnp.int32, (1, NUM_TYPES), 1)
    out_ref[...] = jnp.where(idx == classes, jnp.float32(1.0), jnp.float32(0.0))


def kernel(elem_map, pos):
    del pos
    out_sds = jax.ShapeDtypeStruct((N_NODES * NUM_TYPES,), jnp.float32)
    mesh = plsc.VectorSubcoreMesh(core_axis_name="c", subcore_axis_name="s")
    sc_call = pl.kernel(
        _sc_body,
        out_type=out_sds,
        mesh=mesh,
        compiler_params=pltpu.CompilerParams(needs_layout_passes=False),
        scratch_types=[
            pltpu.VMEM((CHUNK,), jnp.int32),
            pltpu.VMEM((CHUNK * NUM_TYPES,), jnp.float32),
            pltpu.SemaphoreType.DMA,
        ],
    )
    oh2 = sc_call(jnp.reshape(elem_map, (N_NODES,)))

    oh1 = pl.pallas_call(
        _tc_body,
        grid=(N_NODES // ROW_BLOCK,),
        in_specs=[pl.BlockSpec((ROW_BLOCK, 1), lambda i: (i, 0))],
        out_specs=pl.BlockSpec((ROW_BLOCK, NUM_TYPES), lambda i: (i, 0)),
        out_shape=jax.ShapeDtypeStruct((N_NODES, NUM_TYPES), jnp.float32),
    )(elem_map)

    return (oh1, jnp.reshape(oh2, (N_NODES, NUM_TYPES)))


# speedup vs baseline: 1.4223x; 1.4223x over previous
import jax
import jax.numpy as jnp
from jax import lax
from jax.experimental import pallas as pl
from jax.experimental.pallas import tpu as pltpu
from jax.experimental.pallas import tpu_sc as plsc

N_NODES = 100000
NUM_TYPES = 128
L = 16
NW = 32
CHUNK = 384
NFULL = N_NODES // CHUNK
TAIL = N_NODES - NFULL * CHUNK
TAIL_BASE = NFULL * CHUNK
MAX_ITERS = -(-NFULL // NW)
TAIL_WID = 4


def _scatter_groups(buf, idx_v, n_rows, value):
    vals = jnp.full((L,), value, dtype=jnp.float32)
    row_off = lax.broadcasted_iota(jnp.int32, (L,), 0) * NUM_TYPES
    for g in range(n_rows // L):
        cols = idx_v[pl.ds(g * L, L)]
        flat = row_off + (g * L * NUM_TYPES) + cols
        plsc.store_scatter(buf, [flat], vals)


def _sc_body(elem_hbm, out1_hbm, out2_hbm,
             idx0, idx1, buf0, buf1, sem0, sem1):
    wid = lax.axis_index("s") * 2 + lax.axis_index("c")
    idx_s = (idx0, idx1)
    buf_s = (buf0, buf1)
    sem_s = (sem0, sem1)

    zeros = jnp.zeros((L,), jnp.float32)

    def _zero_step(k, _):
        for j in range(8):
            buf0[pl.ds(k * 8 * L + j * L, L)] = zeros
            buf1[pl.ds(k * 8 * L + j * L, L)] = zeros
        return 0

    lax.fori_loop(0, CHUNK * NUM_TYPES // (8 * L), _zero_step, 0)

    def _wait_slot(s):
        pltpu.make_async_copy(buf_s[s], out1_hbm.at[pl.ds(0, CHUNK * NUM_TYPES)],
                              sem_s[s]).wait()
        pltpu.make_async_copy(buf_s[s], out2_hbm.at[pl.ds(0, CHUNK * NUM_TYPES)],
                              sem_s[s]).wait()

    def _fire(s, base, n_rows):
        pltpu.sync_copy(elem_hbm.at[pl.ds(base, n_rows)],
                        idx_s[s].at[pl.ds(0, n_rows)])
        _scatter_groups(buf_s[s], idx_s[s], n_rows, 1.0)
        pltpu.async_copy(buf_s[s].at[pl.ds(0, n_rows * NUM_TYPES)],
                         out1_hbm.at[pl.ds(base * NUM_TYPES, n_rows * NUM_TYPES)],
                         sem_s[s])
        pltpu.async_copy(buf_s[s].at[pl.ds(0, n_rows * NUM_TYPES)],
                         out2_hbm.at[pl.ds(base * NUM_TYPES, n_rows * NUM_TYPES)],
                         sem_s[s])

    for i in range(MAX_ITERS):
        s = i % 2
        if i >= 2:
            c_prev = wid + (i - 2) * NW

            @pl.when(c_prev < NFULL)
            def _(s=s):
                _wait_slot(s)
                _scatter_groups(buf_s[s], idx_s[s], CHUNK, 0.0)

        c_i = wid + i * NW

        @pl.when(c_i < NFULL)
        def _(s=s, c_i=c_i):
            _fire(s, c_i * CHUNK, CHUNK)

    for i in (MAX_ITERS - 2, MAX_ITERS - 1):
        c_i = wid + i * NW

        @pl.when(c_i < NFULL)
        def _(s=i % 2):
            _wait_slot(s)

    @pl.when(wid == TAIL_WID)
    def _():
        _fire(0, TAIL_BASE, TAIL)
        pltpu.make_async_copy(buf0.at[pl.ds(0, TAIL * NUM_TYPES)],
                              out1_hbm.at[pl.ds(0, TAIL * NUM_TYPES)],
                              sem0).wait()
        pltpu.make_async_copy(buf0.at[pl.ds(0, TAIL * NUM_TYPES)],
                              out2_hbm.at[pl.ds(0, TAIL * NUM_TYPES)],
                              sem0).wait()


def kernel(elem_map, pos):
    del pos
    out_sds = jax.ShapeDtypeStruct((N_NODES * NUM_TYPES,), jnp.float32)
    mesh = plsc.VectorSubcoreMesh(core_axis_name="c", subcore_axis_name="s")
    sc_call = pl.kernel(
        _sc_body,
        out_type=(out_sds, out_sds),
        mesh=mesh,
        compiler_params=pltpu.CompilerParams(needs_layout_passes=False),
        scratch_types=[
            pltpu.VMEM((CHUNK,), jnp.int32),
            pltpu.VMEM((CHUNK,), jnp.int32),
            pltpu.VMEM((CHUNK * NUM_TYPES,), jnp.float32),
            pltpu.VMEM((CHUNK * NUM_TYPES,), jnp.float32),
            pltpu.SemaphoreType.DMA,
            pltpu.SemaphoreType.DMA,
        ],
    )
    oh1, oh2 = sc_call(jnp.reshape(elem_map, (N_NODES,)))
    shape2d = (N_NODES, NUM_TYPES)
    return (jnp.reshape(oh1, shape2d), jnp.reshape(oh2, shape2d))


# speedup vs baseline: 1.4942x; 1.0506x over previous
import jax
import jax.numpy as jnp
from jax import lax
from jax.experimental import pallas as pl
from jax.experimental.pallas import tpu as pltpu
from jax.experimental.pallas import tpu_sc as plsc

N_NODES = 100000
NUM_TYPES = 128
L = 16
NW = 32
NSLOTS = 3
CHUNK = 256
NFULL = N_NODES // CHUNK
TAIL = N_NODES - NFULL * CHUNK
TAIL_BASE = NFULL * CHUNK
MAX_ITERS = -(-NFULL // NW)
TAIL_WID = 31


def _scatter_groups(buf, idx_v, n_rows, value):
    vals = jnp.full((L,), value, dtype=jnp.float32)
    row_off = lax.broadcasted_iota(jnp.int32, (L,), 0) * NUM_TYPES
    for g in range(n_rows // L):
        cols = idx_v[pl.ds(g * L, L)]
        flat = row_off + (g * L * NUM_TYPES) + cols
        plsc.store_scatter(buf, [flat], vals)


def _zero_fill(buf, n_words):
    zeros = jnp.zeros((L,), jnp.float32)

    def _step(k, _):
        for j in range(8):
            buf[pl.ds(k * 8 * L + j * L, L)] = zeros
        return 0

    lax.fori_loop(0, n_words // (8 * L), _step, 0)


def _sc_body(elem_hbm, out1_hbm, out2_hbm,
             idx0, idx1, idx2, idxt, buf0, buf1, buf2, buft,
             sem0, sem1, sem2, semt):
    wid = lax.axis_index("s") * 2 + lax.axis_index("c")
    idx_s = (idx0, idx1, idx2)
    buf_s = (buf0, buf1, buf2)
    sem_s = (sem0, sem1, sem2)

    def _wait_slot(s):
        pltpu.make_async_copy(buf_s[s], out1_hbm.at[pl.ds(0, CHUNK * NUM_TYPES)],
                              sem_s[s]).wait()
        pltpu.make_async_copy(buf_s[s], out2_hbm.at[pl.ds(0, CHUNK * NUM_TYPES)],
                              sem_s[s]).wait()

    def _fire(s, base, n_rows):
        pltpu.sync_copy(elem_hbm.at[pl.ds(base, n_rows)],
                        idx_s[s].at[pl.ds(0, n_rows)])
        _scatter_groups(buf_s[s], idx_s[s], n_rows, 1.0)
        pltpu.async_copy(buf_s[s].at[pl.ds(0, n_rows * NUM_TYPES)],
                         out1_hbm.at[pl.ds(base * NUM_TYPES, n_rows * NUM_TYPES)],
                         sem_s[s])
        pltpu.async_copy(buf_s[s].at[pl.ds(0, n_rows * NUM_TYPES)],
                         out2_hbm.at[pl.ds(base * NUM_TYPES, n_rows * NUM_TYPES)],
                         sem_s[s])

    for i in range(NSLOTS):
        _zero_fill(buf_s[i], CHUNK * NUM_TYPES)
        _fire(i, (wid + i * NW) * CHUNK, CHUNK)

    @pl.when(wid == TAIL_WID)
    def _():
        _zero_fill(buft, TAIL * NUM_TYPES)
        pltpu.sync_copy(elem_hbm.at[pl.ds(TAIL_BASE, TAIL)], idxt)
        _scatter_groups(buft, idxt, TAIL, 1.0)
        pltpu.async_copy(buft, out1_hbm.at[pl.ds(TAIL_BASE * NUM_TYPES,
                                                 TAIL * NUM_TYPES)], semt)
        pltpu.async_copy(buft, out2_hbm.at[pl.ds(TAIL_BASE * NUM_TYPES,
                                                 TAIL * NUM_TYPES)], semt)

    for i in range(NSLOTS, MAX_ITERS):
        s = i % NSLOTS
        c_prev = wid + (i - NSLOTS) * NW

        @pl.when(c_prev < NFULL)
        def _(s=s):
            _wait_slot(s)
            _scatter_groups(buf_s[s], idx_s[s], CHUNK, 0.0)

        c_i = wid + i * NW

        @pl.when(c_i < NFULL)
        def _(s=s, c_i=c_i):
            _fire(s, c_i * CHUNK, CHUNK)

    for i in range(MAX_ITERS - NSLOTS, MAX_ITERS):
        c_i = wid + i * NW

        @pl.when(c_i < NFULL)
        def _(s=i % NSLOTS):
            _wait_slot(s)

    @pl.when(wid == TAIL_WID)
    def _():
        pltpu.make_async_copy(buft, out1_hbm.at[pl.ds(0, TAIL * NUM_TYPES)],
                              semt).wait()
        pltpu.make_async_copy(buft, out2_hbm.at[pl.ds(0, TAIL * NUM_TYPES)],
                              semt).wait()


def kernel(elem_map, pos):
    del pos
    out_sds = jax.ShapeDtypeStruct((N_NODES * NUM_TYPES,), jnp.float32)
    mesh = plsc.VectorSubcoreMesh(core_axis_name="c", subcore_axis_name="s")
    sc_call = pl.kernel(
        _sc_body,
        out_type=(out_sds, out_sds),
        mesh=mesh,
        compiler_params=pltpu.CompilerParams(needs_layout_passes=False),
        scratch_types=[
            pltpu.VMEM((CHUNK,), jnp.int32),
            pltpu.VMEM((CHUNK,), jnp.int32),
            pltpu.VMEM((CHUNK,), jnp.int32),
            pltpu.VMEM((TAIL,), jnp.int32),
            pltpu.VMEM((CHUNK * NUM_TYPES,), jnp.float32),
            pltpu.VMEM((CHUNK * NUM_TYPES,), jnp.float32),
            pltpu.VMEM((CHUNK * NUM_TYPES,), jnp.float32),
            pltpu.VMEM((TAIL * NUM_TYPES,), jnp.float32),
            pltpu.SemaphoreType.DMA,
            pltpu.SemaphoreType.DMA,
            pltpu.SemaphoreType.DMA,
            pltpu.SemaphoreType.DMA,
        ],
    )
    oh1, oh2 = sc_call(jnp.reshape(elem_map, (N_NODES,)))
    shape2d = (N_NODES, NUM_TYPES)
    return (jnp.reshape(oh1, shape2d), jnp.reshape(oh2, shape2d))


# speedup vs baseline: 1.5930x; 1.0661x over previous
import jax
import jax.numpy as jnp
from jax import lax
from jax.experimental import pallas as pl
from jax.experimental.pallas import tpu as pltpu
from jax.experimental.pallas import tpu_sc as plsc

N_NODES = 100000
NUM_TYPES = 128
L = 16
NW = 32
NSLOTS = 5
CHUNK = 160
NFULL = N_NODES // CHUNK
MAX_ITERS = -(-NFULL // NW)

TC_BLOCK = 20000


def _scatter_groups(buf, idx_v, n_rows, value):
    vals = jnp.full((L,), value, dtype=jnp.float32)
    row_off = lax.broadcasted_iota(jnp.int32, (L,), 0) * NUM_TYPES
    for g in range(n_rows // L):
        cols = idx_v[pl.ds(g * L, L)]
        flat = row_off + (g * L * NUM_TYPES) + cols
        plsc.store_scatter(buf, [flat], vals)


def _zero_fill(buf, n_words):
    zeros = jnp.zeros((L,), jnp.float32)

    def _step(k, _):
        for j in range(8):
            buf[pl.ds(k * 8 * L + j * L, L)] = zeros
        return 0

    lax.fori_loop(0, n_words // (8 * L), _step, 0)


def _sc_body(elem_hbm, out_hbm, idx0, idx1, idx2, idx3, idx4,
             buf0, buf1, buf2, buf3, buf4, sem0, sem1, sem2, sem3, sem4):
    wid = lax.axis_index("s") * 2 + lax.axis_index("c")
    idx_s = (idx0, idx1, idx2, idx3, idx4)
    buf_s = (buf0, buf1, buf2, buf3, buf4)
    sem_s = (sem0, sem1, sem2, sem3, sem4)

    def _wait_slot(s):
        pltpu.make_async_copy(buf_s[s], out_hbm.at[pl.ds(0, CHUNK * NUM_TYPES)],
                              sem_s[s]).wait()

    def _fire(s, base):
        pltpu.sync_copy(elem_hbm.at[pl.ds(base, CHUNK)], idx_s[s])
        _scatter_groups(buf_s[s], idx_s[s], CHUNK, 1.0)
        pltpu.async_copy(buf_s[s],
                         out_hbm.at[pl.ds(base * NUM_TYPES, CHUNK * NUM_TYPES)],
                         sem_s[s])

    for i in range(NSLOTS):
        _zero_fill(buf_s[i], CHUNK * NUM_TYPES)
        _fire(i, (wid + i * NW) * CHUNK)

    for i in range(NSLOTS, MAX_ITERS):
        s = i % NSLOTS
        _wait_slot(s)
        _scatter_groups(buf_s[s], idx_s[s], CHUNK, 0.0)

        c_i = wid + i * NW

        @pl.when(c_i < NFULL)
        def _(s=s, c_i=c_i):
            _fire(s, c_i * CHUNK)

    for i in range(MAX_ITERS - NSLOTS, MAX_ITERS):
        c_i = wid + i * NW

        @pl.when(c_i < NFULL)
        def _(s=i % NSLOTS):
            _wait_slot(s)


def _tc_body(idx_ref, out_ref):
    idx = jnp.reshape(idx_ref[...], (TC_BLOCK, 1))
    classes = lax.broadcasted_iota(jnp.int32, (1, NUM_TYPES), 1)
    out_ref[...] = jnp.where(idx == classes, jnp.float32(1.0), jnp.float32(0.0))


def kernel(elem_map, pos):
    del pos
    idx_lin = jnp.reshape(elem_map, (N_NODES,))

    out_sds = jax.ShapeDtypeStruct((N_NODES * NUM_TYPES,), jnp.float32)
    mesh = plsc.VectorSubcoreMesh(core_axis_name="c", subcore_axis_name="s")
    sc_call = pl.kernel(
        _sc_body,
        out_type=out_sds,
        mesh=mesh,
        compiler_params=pltpu.CompilerParams(needs_layout_passes=False),
        scratch_types=(
            [pltpu.VMEM((CHUNK,), jnp.int32)] * NSLOTS
            + [pltpu.VMEM((CHUNK * NUM_TYPES,), jnp.float32)] * NSLOTS
            + [pltpu.SemaphoreType.DMA] * NSLOTS
        ),
    )
    oh2 = sc_call(idx_lin)

    idx3d = jnp.reshape(idx_lin, (N_NODES // TC_BLOCK, 1, TC_BLOCK))
    oh1 = pl.pallas_call(
        _tc_body,
        grid=(N_NODES // TC_BLOCK,),
        in_specs=[pl.BlockSpec((1, 1, TC_BLOCK), lambda i: (i, 0, 0))],
        out_specs=pl.BlockSpec((TC_BLOCK, NUM_TYPES), lambda i: (i, 0)),
        out_shape=jax.ShapeDtypeStruct((N_NODES, NUM_TYPES), jnp.float32),
    )(idx3d)

    return (oh1, jnp.reshape(oh2, (N_NODES, NUM_TYPES)))


# speedup vs baseline: 1.6082x; 1.0096x over previous
import jax
import jax.numpy as jnp
from jax import lax
from jax.experimental import pallas as pl
from jax.experimental.pallas import tpu as pltpu
from jax.experimental.pallas import tpu_sc as plsc

N_NODES = 100000
NUM_TYPES = 128
L = 16
NW = 32
NSLOTS = 5
CHUNK = 160
NFULL = N_NODES // CHUNK
MAX_ITERS = -(-NFULL // NW)

TC_BLOCK = 20000


def _scatter_groups(buf, idx_v, n_rows, value):
    vals = jnp.full((L,), value, dtype=jnp.float32)
    row_off = lax.broadcasted_iota(jnp.int32, (L,), 0) * NUM_TYPES
    for g in range(n_rows // L):
        cols = idx_v[pl.ds(g * L, L)]
        flat = row_off + (g * L * NUM_TYPES) + cols
        plsc.store_scatter(buf, [flat], vals)


def _zero_fill(buf, n_words):
    zeros = jnp.zeros((L,), jnp.float32)

    def _step(k, _):
        for j in range(8):
            buf[pl.ds(k * 8 * L + j * L, L)] = zeros
        return 0

    lax.fori_loop(0, n_words // (8 * L), _step, 0)


def _sc_body(elem_hbm, out_hbm, idx0, idx1, idx2, idx3, idx4,
             buf0, buf1, buf2, buf3, buf4, sem0, sem1, sem2, sem3, sem4):
    wid = lax.axis_index("s") * 2 + lax.axis_index("c")
    idx_s = (idx0, idx1, idx2, idx3, idx4)
    buf_s = (buf0, buf1, buf2, buf3, buf4)
    sem_s = (sem0, sem1, sem2, sem3, sem4)

    def _wait_slot(s):
        pltpu.make_async_copy(buf_s[s], out_hbm.at[pl.ds(0, CHUNK * NUM_TYPES)],
                              sem_s[s]).wait()

    def _fire(s, base):
        pltpu.sync_copy(elem_hbm.at[pl.ds(base, CHUNK)], idx_s[s])
        _scatter_groups(buf_s[s], idx_s[s], CHUNK, 1.0)
        pltpu.async_copy(buf_s[s],
                         out_hbm.at[pl.ds(base * NUM_TYPES, CHUNK * NUM_TYPES)],
                         sem_s[s])

    for i in range(NSLOTS):
        _zero_fill(buf_s[i], CHUNK * NUM_TYPES)
        _fire(i, (wid + i * NW) * CHUNK)

    def _round(r, _):
        for j in range(NSLOTS):
            _wait_slot(j)
            _scatter_groups(buf_s[j], idx_s[j], CHUNK, 0.0)
            c_i = wid + (r * NSLOTS + j) * NW

            @pl.when(c_i < NFULL)
            def _(j=j, c_i=c_i):
                _fire(j, c_i * CHUNK)

        return 0

    lax.fori_loop(1, MAX_ITERS // NSLOTS, _round, 0)

    for i in range(MAX_ITERS - NSLOTS, MAX_ITERS):
        c_i = wid + i * NW

        @pl.when(c_i < NFULL)
        def _(s=i % NSLOTS):
            _wait_slot(s)


def _tc_body(idx_ref, out_ref):
    idx = jnp.reshape(idx_ref[...], (TC_BLOCK, 1))
    classes = lax.broadcasted_iota(jnp.int32, (1, NUM_TYPES), 1)
    out_ref[...] = jnp.where(idx == classes, jnp.float32(1.0), jnp.float32(0.0))


def kernel(elem_map, pos):
    del pos
    idx_lin = jnp.reshape(elem_map, (N_NODES,))

    out_sds = jax.ShapeDtypeStruct((N_NODES * NUM_TYPES,), jnp.float32)
    mesh = plsc.VectorSubcoreMesh(core_axis_name="c", subcore_axis_name="s")
    sc_call = pl.kernel(
        _sc_body,
        out_type=out_sds,
        mesh=mesh,
        compiler_params=pltpu.CompilerParams(needs_layout_passes=False),
        scratch_types=(
            [pltpu.VMEM((CHUNK,), jnp.int32)] * NSLOTS
            + [pltpu.VMEM((CHUNK * NUM_TYPES,), jnp.float32)] * NSLOTS
            + [pltpu.SemaphoreType.DMA] * NSLOTS
        ),
    )
    oh2 = sc_call(idx_lin)

    idx3d = jnp.reshape(idx_lin, (N_NODES // TC_BLOCK, 1, TC_BLOCK))
    oh1 = pl.pallas_call(
        _tc_body,
        grid=(N_NODES // TC_BLOCK,),
        in_specs=[pl.BlockSpec((1, 1, TC_BLOCK), lambda i: (i, 0, 0))],
        out_specs=pl.BlockSpec((TC_BLOCK, NUM_TYPES), lambda i: (i, 0)),
        out_shape=jax.ShapeDtypeStruct((N_NODES, NUM_TYPES), jnp.float32),
    )(idx3d)

    return (oh1, jnp.reshape(oh2, (N_NODES, NUM_TYPES)))
